# baseline (speedup 1.0000x reference)
import functools

import jax
import jax.numpy as jnp
from jax import lax
from jax.experimental import pallas as pl
from jax.experimental.pallas import tpu as pltpu

B, L, NL, NF, NV = 16, 128, 40, 32, 20
NW = 32
RPW = (B * NV) // NW


def _fpred_body(lf_ref, vlab_ref, out_ref):
    def body(i, carry):
        oh = (lax.broadcasted_iota(jnp.int32, (NV, L), 1)
              == vlab_ref[i]).astype(jnp.float32)
        out_ref[i] = jnp.dot(oh, lf_ref[i],
                             preferred_element_type=jnp.float32)
        return carry

    lax.fori_loop(0, B, body, 0)


def _loss_body(vlab_ref, fidx_ref, vl_ref, *refs):
    lp_refs = refs[:RPW]
    pool_refs = refs[RPW:2 * RPW]
    fp_ref = refs[2 * RPW]
    out_ref = refs[2 * RPW + 1]
    g = pl.program_id(0)

    @pl.when(g == 0)
    def _init():
        out_ref[...] = jnp.zeros((1, 1), jnp.float32)

    acc = jnp.float32(0.0)
    for k in range(RPW):
        p = g * RPW + k
        x = lp_refs[k][0, 0]
        xmax = jnp.max(x, axis=0, keepdims=True)
        w = jnp.maximum(1.0 - jnp.exp(xmax), 1e-6)
        cand = jnp.where(pool_refs[k][0] == 0, w, 1.0)
        wm = jnp.min(cand, axis=1, keepdims=True)
        m = jnp.log(wm)
        fpk = fp_ref[0, k].reshape(NF, 1)
        s = jnp.sum(jnp.maximum(fpk - m, 0.0))
        maskf = jnp.where(p % NV < vl_ref[p // NV], 1.0, 0.0)
        acc = acc + maskf * s
    out_ref[...] += jnp.full((1, 1), acc, jnp.float32)

    @pl.when(g == NW - 1)
    def _fini():
        tot = lax.fori_loop(0, B, lambda i, a: a + vl_ref[i], 0)
        norm = jnp.maximum(tot, 1).astype(jnp.float32)
        out_ref[...] = out_ref[...] / norm


@jax.jit
def _frame_role_loss(log_pa, v_label, v_l, log_frame, frame_idx, frame_pool):
    vlab = v_label.astype(jnp.int32)
    vlab_flat = vlab.reshape(-1)
    fidx = jnp.take_along_axis(frame_idx.astype(jnp.int32), vlab, axis=1)
    fidx_flat = fidx.reshape(-1)
    vl = v_l.astype(jnp.int32)

    fpred = pl.pallas_call(
        _fpred_body,
        in_specs=[
            pl.BlockSpec((B, L, NF), lambda: (0, 0, 0)),
            pl.BlockSpec((B, NV, 1), lambda: (0, 0, 0)),
        ],
        out_shape=jax.ShapeDtypeStruct((B, NV, NF), jnp.float32),
        out_specs=pl.BlockSpec((B, NV, NF), lambda: (0, 0, 0)),
    )(log_frame, vlab.reshape(B, NV, 1))

    lp_specs = [
        pl.BlockSpec((1, 1, L, NL),
                     functools.partial(
                         lambda k, g, vlab, fidx, vl:
                         ((g * RPW + k) // NV, vlab[g * RPW + k], 0, 0), k))
        for k in range(RPW)
    ]
    pool_specs = [
        pl.BlockSpec((1, NF, NL),
                     functools.partial(
                         lambda k, g, vlab, fidx, vl:
                         (fidx[g * RPW + k], 0, 0), k))
        for k in range(RPW)
    ]
    fp_spec = pl.BlockSpec((1, RPW, NF),
                           lambda g, vlab, fidx, vl: (g, 0, 0))

    grid_spec = pltpu.PrefetchScalarGridSpec(
        num_scalar_prefetch=3,
        grid=(NW,),
        in_specs=lp_specs + pool_specs + [fp_spec],
        out_specs=pl.BlockSpec((1, 1), lambda g, vlab, fidx, vl: (0, 0)),
    )
    out = pl.pallas_call(
        _loss_body,
        grid_spec=grid_spec,
        out_shape=jax.ShapeDtypeStruct((1, 1), jnp.float32),
    )(vlab_flat, fidx_flat, vl,
      *([log_pa] * RPW), *([frame_pool] * RPW), fpred.reshape(NW, RPW, NF))
    return out.reshape(())


def kernel(log_pa, score, v_label, v_l, role_label, roleset_id, log_frame,
           frame_idx, frame_pool):
    return _frame_role_loss(log_pa, v_label, v_l, log_frame, frame_idx,
                            frame_pool)
